# baseline (speedup 1.0000x reference)
import functools

import jax
import jax.numpy as jnp
from jax import lax
from jax.experimental import pallas as pl
from jax.experimental.pallas import tpu as pltpu
from jax.experimental.pallas import tpu_sc as plsc


@functools.lru_cache(maxsize=None)
def _make_gather(n_tok, s, d):
    info = plsc.get_sparse_core_info()
    num_cores, num_subcores = info.num_cores, info.num_subcores
    nw = num_cores * num_subcores
    chunks = n_tok // nw
    assert chunks * nw == n_tok
    assert chunks % 2 == 0 and chunks >= 4
    s_pad = ((s + 7) // 8) * 8
    assert s_pad <= 128
    s_main = (s // 8) * 8
    s_tail = s - s_main
    idx_minor = ((s_pad + 15) // 16) * 16

    mesh = plsc.VectorSubcoreMesh(core_axis_name="c", subcore_axis_name="s")

    @functools.partial(
        pl.kernel,
        mesh=mesh,
        out_type=jax.ShapeDtypeStruct((n_tok, s, d), jnp.float32),
        scratch_types=[
            pltpu.VMEM((chunks, idx_minor), jnp.int32),
            pltpu.VMEM((2, s_pad, d), jnp.float32),
            pltpu.SemaphoreType.DMA,
            pltpu.SemaphoreType.DMA,
            pltpu.SemaphoreType.DMA,
            pltpu.SemaphoreType.DMA,
        ],
    )
    def k(table_hbm, x_hbm, out_hbm, idx_v, rows_v, g0, g1, o0, o1):
        wid = lax.axis_index("s") * num_cores + lax.axis_index("c")
        base_tok = wid * chunks
        pltpu.sync_copy(x_hbm.at[pl.ds(base_tok, chunks)], idx_v)

        gsem = (g0, g1)
        osem = (o0, o1)

        def gather(c, b, sem):
            return pltpu.make_async_copy(
                table_hbm.at[idx_v.at[c, pl.ds(0, s_pad)]], rows_v.at[b], sem)

        def out_parts(c, b, sem):
            dst = out_hbm.at[base_tok + c]
            parts = [pltpu.make_async_copy(
                rows_v.at[b, pl.ds(0, s_main)], dst.at[pl.ds(0, s_main)], sem)]
            if s_tail:
                parts.append(pltpu.make_async_copy(
                    rows_v.at[b, pl.ds(s_main, s_tail)],
                    dst.at[pl.ds(s_main, s_tail)], sem))
            return parts

        def out_start(c, b, sem):
            for p in out_parts(c, b, sem):
                p.start()

        def out_wait(c, b, sem):
            for p in out_parts(c, b, sem):
                p.wait()

        gather(0, 0, g0).start()
        gather(0, 0, g0).wait()
        out_start(0, 0, o0)
        gather(1, 1, g1).start()

        def step(c, b):
            gather(c, b, gsem[b]).wait()
            out_start(c, b, osem[b])
            out_wait(c - 1, b ^ 1, osem[b ^ 1])
            gather(c + 1, b ^ 1, gsem[b ^ 1]).start()

        def body(g, carry):
            step(2 * g + 1, 1)
            step(2 * g + 2, 0)
            return carry

        lax.fori_loop(0, (chunks - 2) // 2, body, 0)

        last = chunks - 1
        gather(last, 1, g1).wait()
        out_start(last, 1, o1)
        out_wait(last - 1, 0, o0)
        out_wait(last, 1, o1)

    return k


def kernel(x, table):
    n_tok, s = x.shape
    d = table.shape[1]
    idx_minor = ((((s + 7) // 8) * 8 + 15) // 16) * 16
    xp = jnp.pad(x.astype(jnp.int32), ((0, 0), (0, idx_minor - s)))
    return _make_gather(n_tok, s, d)(table, xp)
